# baseline (speedup 1.0000x reference)
import jax
import jax.numpy as jnp
from jax.experimental import pallas as pl
from jax.experimental.pallas import tpu as pltpu

H = W_DIM = 56
HP = H + 2
NPIX = HP * HP
MARGIN = 64
XLEN = MARGIN + NPIX + 92


def _conv_body(x_ref, w_ref, b_ref, o_ref):
    acc = None
    for k in range(9):
        kh, kw = k // 3, k % 3
        dk = (kh - 1) * HP + (kw - 1)
        xs = x_ref[0, :, MARGIN + dk : MARGIN + dk + NPIX]
        d = jax.lax.dot_general(
            w_ref[k], xs, (((1,), (0,)), ((), ())),
            preferred_element_type=jnp.float32,
        )
        acc = d if acc is None else acc + d
    o_ref[0] = acc + b_ref[...]


def kernel(x, W, b):
    n, c, h, w = x.shape
    o = W.shape[0]
    xp = jnp.pad(x, ((0, 0), (0, 0), (1, 1), (1, 1))).reshape(n, c, NPIX)
    xf = jnp.pad(xp, ((0, 0), (0, 0), (MARGIN, XLEN - MARGIN - NPIX)))
    wr = jnp.transpose(W, (2, 3, 0, 1)).reshape(9, o, c)
    b2 = b.reshape(o, 1)

    out_flat = pl.pallas_call(
        _conv_body,
        out_shape=jax.ShapeDtypeStruct((n, o, NPIX), jnp.float32),
        grid=(n,),
        in_specs=[
            pl.BlockSpec((1, c, XLEN), lambda i: (i, 0, 0)),
            pl.BlockSpec((9, o, c), lambda i: (0, 0, 0)),
            pl.BlockSpec((o, 1), lambda i: (0, 0)),
        ],
        out_specs=pl.BlockSpec((1, o, NPIX), lambda i: (i, 0, 0)),
        compiler_params=pltpu.CompilerParams(
            dimension_semantics=("parallel",),
        ),
        name="conv3x3_flat",
    )(xf, wr, b2)

    return out_flat.reshape(n, o, HP, HP)[:, :, 1:57, 1:57]


# speedup vs baseline: 1.1426x; 1.1426x over previous
import jax
import jax.numpy as jnp
from jax.experimental import pallas as pl
from jax.experimental.pallas import tpu as pltpu

H = 56
HP = H + 2
NPIX = HP * HP
MARGIN = 64
XLEN = MARGIN + NPIX + 92
OFFS = [(kh - 1) * HP + (kw - 1) for kh in range(3) for kw in range(3)]


def _conv_body(x_ref, w_ref, b_ref, o_ref):
    acc = None
    for p in range(5):
        ta, tb = 2 * p, 2 * p + 1
        da = OFFS[ta]
        xa = x_ref[0, :, MARGIN + da : MARGIN + da + NPIX]
        if tb < 9:
            db = OFFS[tb]
            xb = x_ref[0, :, MARGIN + db : MARGIN + db + NPIX]
        else:
            xb = xa
        xcat = jnp.concatenate([xa, xb], axis=0)
        d = jax.lax.dot_general(
            w_ref[p], xcat, (((1,), (0,)), ((), ())),
            preferred_element_type=jnp.float32,
        )
        acc = d if acc is None else acc + d
    o_ref[0] = acc + b_ref[...]


def kernel(x, W, b):
    n, c, h, w = x.shape
    o = W.shape[0]
    xp = jnp.pad(x, ((0, 0), (0, 0), (1, 1), (1, 1))).reshape(n, c, NPIX)
    xf = jnp.pad(xp, ((0, 0), (0, 0), (MARGIN, XLEN - MARGIN - NPIX)))
    xf = xf.astype(jnp.bfloat16)
    wr = jnp.transpose(W, (2, 3, 0, 1)).reshape(9, o, c)
    wr = jnp.concatenate([wr, jnp.zeros((1, o, c), wr.dtype)], axis=0)
    wp = wr.reshape(5, 2, o, c).transpose(0, 2, 1, 3).reshape(5, o, 2 * c)
    wp = wp.astype(jnp.bfloat16)
    b2 = b.reshape(o, 1)

    out_flat = pl.pallas_call(
        _conv_body,
        out_shape=jax.ShapeDtypeStruct((n, o, NPIX), jnp.float32),
        grid=(n,),
        in_specs=[
            pl.BlockSpec((1, c, XLEN), lambda i: (i, 0, 0)),
            pl.BlockSpec((5, o, 2 * c), lambda i: (0, 0, 0)),
            pl.BlockSpec((o, 1), lambda i: (0, 0)),
        ],
        out_specs=pl.BlockSpec((1, o, NPIX), lambda i: (i, 0, 0)),
        compiler_params=pltpu.CompilerParams(
            dimension_semantics=("parallel",),
        ),
        name="conv3x3_flat",
    )(xf, wp, b2)

    return out_flat.reshape(n, o, HP, HP)[:, :, 1:57, 1:57]


# speedup vs baseline: 1.7864x; 1.5635x over previous
import jax
import jax.numpy as jnp
from jax.experimental import pallas as pl
from jax.experimental.pallas import tpu as pltpu

H = 56
NVALID = H * H
MARGIN = 128
XLEN = MARGIN + NVALID + MARGIN


def _conv_body(x_ref, w_ref, m_ref, b_ref, o_ref, xq_ref):
    @pl.when(pl.program_id(0) == 0)
    def _():
        xq_ref[:, :MARGIN] = jnp.zeros((128, MARGIN), jnp.bfloat16)
        xq_ref[:, MARGIN + NVALID:] = jnp.zeros((128, MARGIN), jnp.bfloat16)

    xq_ref[:, MARGIN:MARGIN + NVALID] = x_ref[0].astype(jnp.bfloat16)

    acc = None
    for p in range(5):
        halves = []
        for t in (2 * p, 2 * p + 1):
            if t < 9:
                kh, kw = t // 3, t % 3
                off = MARGIN - 57 + kw + kh * 56
                xs = xq_ref[:, off:off + NVALID]
                if kw == 0:
                    xs = xs * m_ref[0]
                elif kw == 2:
                    xs = xs * m_ref[1]
            else:
                xs = jnp.zeros((128, NVALID), jnp.bfloat16)
            halves.append(xs)
        xcat = jnp.concatenate(halves, axis=0)
        d = jax.lax.dot_general(
            w_ref[p], xcat, (((1,), (0,)), ((), ())),
            preferred_element_type=jnp.float32,
        )
        acc = d if acc is None else acc + d
    o_ref[0] = acc + b_ref[...]


def kernel(x, W, b):
    n, c, h, w = x.shape
    o = W.shape[0]
    xf = x.reshape(n, c, NVALID)
    wr = jnp.transpose(W, (2, 3, 0, 1)).reshape(9, o, c)
    wr = jnp.concatenate([wr, jnp.zeros((1, o, c), wr.dtype)], axis=0)
    wp = wr.reshape(5, 2, o, c).transpose(0, 2, 1, 3).reshape(5, o, 2 * c)
    wp = wp.astype(jnp.bfloat16)
    b2 = b.reshape(o, 1)
    j = jnp.arange(NVALID)
    masks = jnp.stack([(j % H) != 0, (j % H) != (H - 1)])
    masks = jnp.broadcast_to(masks[:, None, :], (2, c, NVALID))
    masks = masks.astype(jnp.bfloat16)

    out_flat = pl.pallas_call(
        _conv_body,
        out_shape=jax.ShapeDtypeStruct((n, o, NVALID), jnp.float32),
        grid=(n,),
        in_specs=[
            pl.BlockSpec((1, c, NVALID), lambda i: (i, 0, 0)),
            pl.BlockSpec((5, o, 2 * c), lambda i: (0, 0, 0)),
            pl.BlockSpec((2, c, NVALID), lambda i: (0, 0, 0)),
            pl.BlockSpec((o, 1), lambda i: (0, 0)),
        ],
        out_specs=pl.BlockSpec((1, o, NVALID), lambda i: (i, 0, 0)),
        scratch_shapes=[pltpu.VMEM((c, XLEN), jnp.bfloat16)],
        compiler_params=pltpu.CompilerParams(
            dimension_semantics=("parallel",),
        ),
        name="conv3x3_flat",
    )(xf, wp, masks, b2)

    return out_flat.reshape(n, o, H, H)
